# Initial kernel scaffold; baseline (speedup 1.0000x reference)
import jax
import jax.numpy as jnp
from jax.experimental import pallas as pl


def kernel(idx_emb1, idx_embbag1, emb1_w, embbag1_w):
    raise NotImplementedError("write your pallas kernel here")


# speedup vs baseline: 11.3022x; 11.3022x over previous
import functools

import jax
import jax.numpy as jnp
from jax import lax
from jax.experimental import pallas as pl
from jax.experimental.pallas import tpu as pltpu
from jax.experimental.pallas import tpu_sc as plsc

_B = 16384
_L = 20
_D1 = 3
_D2 = 32
_V1 = 100
_V2 = 200
_OUT = _L * _D1 + _D2
_NC = 2
_NS = 16
_NW = _NC * _NS
_R = _B // _NW
_G = _R // 16

_mesh = plsc.VectorSubcoreMesh(core_axis_name="c", subcore_axis_name="s")


def _body(idx1_hbm, idx2_hbm, t1_hbm, t2_hbm, out_hbm,
          idx1_v, idx2_v, t1_v, t2_v, out_v, sem):
    wid = lax.axis_index("s") * _NC + lax.axis_index("c")
    base = wid * _R

    c1 = pltpu.async_copy(idx1_hbm.at[pl.ds(base * _L, _R * _L)], idx1_v, sem)
    c2 = pltpu.async_copy(idx2_hbm.at[pl.ds(base * _L, _R * _L)], idx2_v, sem)
    c3 = pltpu.async_copy(t1_hbm, t1_v, sem)
    c4 = pltpu.async_copy(t2_hbm, t2_v, sem)
    c1.wait()
    c2.wait()
    c3.wait()
    c4.wait()

    inv_l = jnp.float32(1.0 / _L)

    def group(g, carry):
        rows = g * 16 + lax.iota(jnp.int32, 16)
        i1base = rows * _L
        obase = rows * _OUT
        for l in range(_L):
            iv = plsc.load_gather(idx1_v, [i1base + l])
            for c in range(_D1):
                vals = plsc.load_gather(t1_v, [iv * _D1 + c])
                plsc.store_scatter(out_v, [obase + (l * _D1 + c)], vals)
        acc = [jnp.zeros((16,), jnp.float32) for _ in range(_D2)]
        for l in range(_L):
            iv = plsc.load_gather(idx2_v, [i1base + l])
            ivd = iv * _D2
            for d in range(_D2):
                acc[d] = acc[d] + plsc.load_gather(t2_v, [ivd + d])
        for d in range(_D2):
            plsc.store_scatter(out_v, [obase + (_L * _D1 + d)], acc[d] * inv_l)
        return carry

    lax.fori_loop(0, _G, group, 0)
    pltpu.sync_copy(out_v, out_hbm.at[pl.ds(base * _OUT, _R * _OUT)])


_run = functools.partial(
    pl.kernel,
    out_type=jax.ShapeDtypeStruct((_B * _OUT,), jnp.float32),
    mesh=_mesh,
    compiler_params=pltpu.CompilerParams(needs_layout_passes=False),
    scratch_types=[
        pltpu.VMEM((_R * _L,), jnp.int32),
        pltpu.VMEM((_R * _L,), jnp.int32),
        pltpu.VMEM((_V1 * _D1,), jnp.float32),
        pltpu.VMEM((_V2 * _D2,), jnp.float32),
        pltpu.VMEM((_R * _OUT,), jnp.float32),
        pltpu.SemaphoreType.DMA,
    ],
)(_body)


@jax.jit
def kernel(idx_emb1, idx_embbag1, emb1_w, embbag1_w):
    out = _run(idx_emb1.astype(jnp.int32).reshape(-1),
               idx_embbag1.astype(jnp.int32).reshape(-1),
               emb1_w.reshape(-1), embbag1_w.reshape(-1))
    return out.reshape(_B, _OUT)


# speedup vs baseline: 35.0938x; 3.1051x over previous
import functools

import jax
import jax.numpy as jnp
from jax import lax
from jax.experimental import pallas as pl
from jax.experimental.pallas import tpu as pltpu
from jax.experimental.pallas import tpu_sc as plsc

_B = 16384
_L = 20
_D1 = 3
_D2 = 32
_V1 = 100
_V2 = 200
_OUT = _L * _D1 + _D2
_NC = 2
_NS = 16
_NW = _NC * _NS
_R = _B // _NW
_G = _R // 16

_mesh = plsc.VectorSubcoreMesh(core_axis_name="c", subcore_axis_name="s")


def _body(idx1_hbm, idx2_hbm, t1_hbm, t2_hbm, out_hbm,
          idx1_v, idx2_v, t1_v, t2_v, out_v, sem):
    wid = lax.axis_index("s") * _NC + lax.axis_index("c")
    base = wid * _R

    c1 = pltpu.async_copy(idx1_hbm.at[pl.ds(base * _L, _R * _L)], idx1_v, sem)
    c2 = pltpu.async_copy(idx2_hbm.at[pl.ds(base * _L, _R * _L)], idx2_v, sem)
    c3 = pltpu.async_copy(t1_hbm, t1_v, sem)
    c4 = pltpu.async_copy(t2_hbm, t2_v, sem)
    c1.wait()
    c2.wait()
    c3.wait()
    c4.wait()

    inv_l = jnp.float32(1.0 / _L)

    @functools.partial(plsc.parallel_loop, 0, _G, unroll=2)
    def group(g):
        rows = g * 16 + lax.iota(jnp.int32, 16)
        i1base = rows * _L
        obase = rows * _OUT
        for l in range(_L):
            iv = plsc.load_gather(idx1_v, [i1base + l])
            for c in range(_D1):
                vals = plsc.load_gather(t1_v, [iv * _D1 + c])
                plsc.store_scatter(out_v, [obase + (l * _D1 + c)], vals)
        acc = [jnp.zeros((16,), jnp.float32) for _ in range(_D2)]
        for l in range(_L):
            iv = plsc.load_gather(idx2_v, [i1base + l])
            ivd = iv * _D2
            for d in range(_D2):
                acc[d] = acc[d] + plsc.load_gather(t2_v, [ivd + d])
        for d in range(_D2):
            plsc.store_scatter(out_v, [obase + (_L * _D1 + d)], acc[d] * inv_l)

    pltpu.sync_copy(out_v, out_hbm.at[pl.ds(base * _OUT, _R * _OUT)])


_run = functools.partial(
    pl.kernel,
    out_type=jax.ShapeDtypeStruct((_B * _OUT,), jnp.float32),
    mesh=_mesh,
    compiler_params=pltpu.CompilerParams(needs_layout_passes=False),
    scratch_types=[
        pltpu.VMEM((_R * _L,), jnp.int32),
        pltpu.VMEM((_R * _L,), jnp.int32),
        pltpu.VMEM((_V1 * _D1,), jnp.float32),
        pltpu.VMEM((_V2 * _D2,), jnp.float32),
        pltpu.VMEM((_R * _OUT,), jnp.float32),
        pltpu.SemaphoreType.DMA,
    ],
)(_body)


@jax.jit
def kernel(idx_emb1, idx_embbag1, emb1_w, embbag1_w):
    out = _run(idx_emb1.astype(jnp.int32).reshape(-1),
               idx_embbag1.astype(jnp.int32).reshape(-1),
               emb1_w.reshape(-1), embbag1_w.reshape(-1))
    return out.reshape(_B, _OUT)


# speedup vs baseline: 35.1705x; 1.0022x over previous
import functools

import jax
import jax.numpy as jnp
from jax import lax
from jax.experimental import pallas as pl
from jax.experimental.pallas import tpu as pltpu
from jax.experimental.pallas import tpu_sc as plsc

_B = 16384
_L = 20
_D1 = 3
_D2 = 32
_V1 = 100
_V2 = 200
_OUT = _L * _D1 + _D2
_NC = 2
_NS = 16
_NW = _NC * _NS
_R = _B // _NW
_G = _R // 16

_mesh = plsc.VectorSubcoreMesh(core_axis_name="c", subcore_axis_name="s")


def _body(idx1_hbm, idx2_hbm, t1_hbm, t2_hbm, out_hbm,
          idx1_v, idx2_v, t1_v, t2_v, out_v, sem):
    wid = lax.axis_index("s") * _NC + lax.axis_index("c")
    base = wid * _R

    c1 = pltpu.async_copy(idx1_hbm.at[pl.ds(base * _L, _R * _L)], idx1_v, sem)
    c2 = pltpu.async_copy(idx2_hbm.at[pl.ds(base * _L, _R * _L)], idx2_v, sem)
    c3 = pltpu.async_copy(t1_hbm, t1_v, sem)
    c4 = pltpu.async_copy(t2_hbm, t2_v, sem)
    c1.wait()
    c2.wait()
    c3.wait()
    c4.wait()

    inv_l = jnp.float32(1.0 / _L)

    @functools.partial(plsc.parallel_loop, 0, _G, unroll=1)
    def group(g):
        rows = g * 16 + lax.iota(jnp.int32, 16)
        i1base = rows * _L
        obase = rows * _OUT
        for l in range(_L):
            iv = plsc.load_gather(idx1_v, [i1base + l])
            for c in range(_D1):
                vals = plsc.load_gather(t1_v, [iv * _D1 + c])
                plsc.store_scatter(out_v, [obase + (l * _D1 + c)], vals)
        acc = [jnp.zeros((16,), jnp.float32) for _ in range(_D2)]
        for l in range(_L):
            iv = plsc.load_gather(idx2_v, [i1base + l])
            ivd = iv * _D2
            for d in range(_D2):
                acc[d] = acc[d] + plsc.load_gather(t2_v, [ivd + d])
        for d in range(_D2):
            plsc.store_scatter(out_v, [obase + (_L * _D1 + d)], acc[d] * inv_l)

    pltpu.sync_copy(out_v, out_hbm.at[pl.ds(base * _OUT, _R * _OUT)])


_run = functools.partial(
    pl.kernel,
    out_type=jax.ShapeDtypeStruct((_B * _OUT,), jnp.float32),
    mesh=_mesh,
    compiler_params=pltpu.CompilerParams(needs_layout_passes=False),
    scratch_types=[
        pltpu.VMEM((_R * _L,), jnp.int32),
        pltpu.VMEM((_R * _L,), jnp.int32),
        pltpu.VMEM((_V1 * _D1,), jnp.float32),
        pltpu.VMEM((_V2 * _D2,), jnp.float32),
        pltpu.VMEM((_R * _OUT,), jnp.float32),
        pltpu.SemaphoreType.DMA,
    ],
)(_body)


@jax.jit
def kernel(idx_emb1, idx_embbag1, emb1_w, embbag1_w):
    out = _run(idx_emb1.astype(jnp.int32).reshape(-1),
               idx_embbag1.astype(jnp.int32).reshape(-1),
               emb1_w.reshape(-1), embbag1_w.reshape(-1))
    return out.reshape(_B, _OUT)
